# Initial kernel scaffold; baseline (speedup 1.0000x reference)
import jax
import jax.numpy as jnp
from jax.experimental import pallas as pl


def kernel(x, start_w, start_b, w_gate, W1, b1, W2, b2):
    raise NotImplementedError("write your pallas kernel here")


# speedup vs baseline: 1.1545x; 1.1545x over previous
import functools

import jax
import jax.numpy as jnp
import numpy as np
from jax.experimental import pallas as pl
from jax.experimental.pallas import tpu as pltpu

B, L, N, D = 32, 96, 16, 64
E, K = 8, 2
D_FF = 128
LN = L * N
ND = N * D
EPS = float(np.finfo(float).eps)


def _gate_body(x_ref, wv_ref, sb_ref, wg_ref, idx_ref, gat_ref):
    xb = x_ref[0]
    s = jnp.sum(xb * wv_ref[...], axis=1, keepdims=True) + sb_ref[...]
    logits = jnp.sum(s * wg_ref[...], axis=0, keepdims=True)
    iota = jax.lax.broadcasted_iota(jnp.int32, (1, E), 1)
    m1 = jnp.max(logits, axis=1, keepdims=True)
    i1 = jnp.min(jnp.where(logits == m1, iota, E), axis=1, keepdims=True)
    l2 = jnp.where(iota == i1, -jnp.inf, logits)
    m2 = jnp.max(l2, axis=1, keepdims=True)
    i2 = jnp.min(jnp.where(l2 == m2, iota, E), axis=1, keepdims=True)
    r = jnp.exp(m2 - m1)
    g1 = 1.0 / (1.0 + r)
    g2 = r / (1.0 + r)
    idx_ref[0] = jnp.concatenate([i1, i2], axis=1)
    gat_ref[0] = jnp.concatenate([g1, g2], axis=1)


def _expert_body(idx_ref, gat_ref, x_ref, w1_ref, b1_ref, w2_ref, b2_ref,
                 o_ref):
    b = pl.program_id(0)
    k = pl.program_id(1)
    xm = x_ref[0]
    h = jnp.dot(xm, w1_ref[0], preferred_element_type=jnp.float32)
    h = jnp.maximum(h + b1_ref[0], 0.0)
    y = jnp.dot(h, w2_ref[0], preferred_element_type=jnp.float32)
    y = y + b2_ref[0]
    g = gat_ref[b * K + k]
    contrib = g * jnp.exp(y)

    @pl.when(k == 0)
    def _():
        o_ref[0] = contrib

    @pl.when(k == K - 1)
    def _():
        acc = o_ref[0] + contrib
        acc = jnp.where(acc == 0.0, EPS, acc)
        o_ref[0] = jnp.log(acc)


@jax.jit
def kernel(x, start_w, start_b, w_gate, W1, b1, W2, b2):
    x2 = x.reshape(B, L, ND)
    x3 = x.reshape(B, LN, D)
    wv = jnp.tile(start_w[:, 0] / N, (N,)).reshape(1, ND)
    sb = start_b.reshape(1, 1)

    idx, gates = pl.pallas_call(
        _gate_body,
        grid=(B,),
        in_specs=[
            pl.BlockSpec((1, L, ND), lambda b: (b, 0, 0)),
            pl.BlockSpec((1, ND), lambda b: (0, 0)),
            pl.BlockSpec((1, 1), lambda b: (0, 0)),
            pl.BlockSpec((L, E), lambda b: (0, 0)),
        ],
        out_specs=[
            pl.BlockSpec((1, 1, K), lambda b: (b, 0, 0)),
            pl.BlockSpec((1, 1, K), lambda b: (b, 0, 0)),
        ],
        out_shape=[
            jax.ShapeDtypeStruct((B, 1, K), jnp.int32),
            jax.ShapeDtypeStruct((B, 1, K), jnp.float32),
        ],
    )(x2, wv, sb, w_gate)

    idx_flat = idx.reshape(B * K)
    gat_flat = gates.reshape(B * K)

    out = pl.pallas_call(
        _expert_body,
        grid_spec=pltpu.PrefetchScalarGridSpec(
            num_scalar_prefetch=2,
            grid=(B, K),
            in_specs=[
                pl.BlockSpec((1, LN, D), lambda b, k, i, g: (b, 0, 0)),
                pl.BlockSpec((1, D, D_FF), lambda b, k, i, g: (i[b * K + k], 0, 0)),
                pl.BlockSpec((1, 1, D_FF), lambda b, k, i, g: (i[b * K + k], 0, 0)),
                pl.BlockSpec((1, D_FF, D), lambda b, k, i, g: (i[b * K + k], 0, 0)),
                pl.BlockSpec((1, 1, D), lambda b, k, i, g: (i[b * K + k], 0, 0)),
            ],
            out_specs=pl.BlockSpec((1, LN, D), lambda b, k, i, g: (b, 0, 0)),
        ),
        out_shape=jax.ShapeDtypeStruct((B, LN, D), jnp.float32),
    )(idx_flat, gat_flat, x3, W1, b1.reshape(E, 1, D_FF), W2,
      b2.reshape(E, 1, D))

    return out.reshape(B, L, N, D)


# speedup vs baseline: 1.3228x; 1.1458x over previous
import jax
import jax.numpy as jnp
import numpy as np
from jax.experimental import pallas as pl
from jax.experimental.pallas import tpu as pltpu

B, L, N, D = 32, 96, 16, 64
E, K = 8, 2
D_FF = 128
LN = L * N
EPS = float(np.finfo(float).eps)


def _gate_body(x_ref, wv_ref, sb_ref, wg_ref, idx_ref, gat_ref):
    xb = x_ref[0]
    t = jnp.sum(xb * wv_ref[...], axis=(1, 2))
    s = t.reshape(L, 1) + sb_ref[...]
    logits = jnp.sum(s * wg_ref[...], axis=0, keepdims=True)
    iota = jax.lax.broadcasted_iota(jnp.int32, (1, E), 1)
    m1 = jnp.max(logits, axis=1, keepdims=True)
    i1 = jnp.min(jnp.where(logits == m1, iota, E), axis=1, keepdims=True)
    l2 = jnp.where(iota == i1, -jnp.inf, logits)
    m2 = jnp.max(l2, axis=1, keepdims=True)
    i2 = jnp.min(jnp.where(l2 == m2, iota, E), axis=1, keepdims=True)
    r = jnp.exp(m2 - m1)
    g1 = 1.0 / (1.0 + r)
    g2 = r / (1.0 + r)
    idx_ref[0] = jnp.concatenate([i1, i2], axis=1)
    gat_ref[0] = jnp.concatenate([g1, g2], axis=1)


def _expert_body(idx_ref, gat_ref, x_ref, w1a_ref, w1b_ref, b1a_ref,
                 b1b_ref, w2a_ref, w2b_ref, b2a_ref, b2b_ref, o_ref,
                 w2d_ref):
    b = pl.program_id(0)
    xm = x_ref[0].reshape(LN, D)
    w1 = jnp.concatenate([w1a_ref[0], w1b_ref[0]], axis=1)
    bias1 = jnp.concatenate([b1a_ref[0], b1b_ref[0]], axis=1)
    h = jnp.dot(xm, w1, preferred_element_type=jnp.float32)
    h = jnp.maximum(h + bias1, 0.0)
    w2d_ref[:D_FF, :D] = w2a_ref[0]
    w2d_ref[:D_FF, D:] = jnp.zeros((D_FF, D), jnp.float32)
    w2d_ref[D_FF:, :D] = jnp.zeros((D_FF, D), jnp.float32)
    w2d_ref[D_FF:, D:] = w2b_ref[0]
    bias2 = jnp.concatenate([b2a_ref[0], b2b_ref[0]], axis=1)
    y = jnp.dot(h, w2d_ref[...], preferred_element_type=jnp.float32)
    y = y + bias2
    ga = gat_ref[K * b]
    gb = gat_ref[K * b + 1]
    ey = jnp.exp(y)
    acc = ga * ey[:, :D] + gb * ey[:, D:]
    acc = jnp.where(acc == 0.0, EPS, acc)
    o_ref[0] = jnp.log(acc).reshape(L, N, D)


@jax.jit
def kernel(x, start_w, start_b, w_gate, W1, b1, W2, b2):
    wv = (start_w[:, 0] / N).reshape(1, 1, D)
    sb = start_b.reshape(1, 1)

    idx, gates = pl.pallas_call(
        _gate_body,
        grid=(B,),
        in_specs=[
            pl.BlockSpec((1, L, N, D), lambda b: (b, 0, 0, 0)),
            pl.BlockSpec((1, 1, D), lambda b: (0, 0, 0)),
            pl.BlockSpec((1, 1), lambda b: (0, 0)),
            pl.BlockSpec((L, E), lambda b: (0, 0)),
        ],
        out_specs=[
            pl.BlockSpec((1, 1, K), lambda b: (b, 0, 0)),
            pl.BlockSpec((1, 1, K), lambda b: (b, 0, 0)),
        ],
        out_shape=[
            jax.ShapeDtypeStruct((B, 1, K), jnp.int32),
            jax.ShapeDtypeStruct((B, 1, K), jnp.float32),
        ],
    )(x, wv, sb, w_gate)

    idx_flat = idx.reshape(B * K)
    gat_flat = gates.reshape(B * K)

    b1r = b1.reshape(E, 1, D_FF)
    b2r = b2.reshape(E, 1, D)
    out = pl.pallas_call(
        _expert_body,
        grid_spec=pltpu.PrefetchScalarGridSpec(
            num_scalar_prefetch=2,
            grid=(B,),
            in_specs=[
                pl.BlockSpec((1, L, N, D), lambda b, i, g: (b, 0, 0, 0)),
                pl.BlockSpec((1, D, D_FF), lambda b, i, g: (i[K * b], 0, 0)),
                pl.BlockSpec((1, D, D_FF), lambda b, i, g: (i[K * b + 1], 0, 0)),
                pl.BlockSpec((1, 1, D_FF), lambda b, i, g: (i[K * b], 0, 0)),
                pl.BlockSpec((1, 1, D_FF), lambda b, i, g: (i[K * b + 1], 0, 0)),
                pl.BlockSpec((1, D_FF, D), lambda b, i, g: (i[K * b], 0, 0)),
                pl.BlockSpec((1, D_FF, D), lambda b, i, g: (i[K * b + 1], 0, 0)),
                pl.BlockSpec((1, 1, D), lambda b, i, g: (i[K * b], 0, 0)),
                pl.BlockSpec((1, 1, D), lambda b, i, g: (i[K * b + 1], 0, 0)),
            ],
            out_specs=pl.BlockSpec((1, L, N, D), lambda b, i, g: (b, 0, 0, 0)),
            scratch_shapes=[pltpu.VMEM((2 * D_FF, 2 * D), jnp.float32)],
        ),
        out_shape=jax.ShapeDtypeStruct((B, L, N, D), jnp.float32),
    )(idx_flat, gat_flat, x, W1, W1, b1r, b1r, W2, W2, b2r, b2r)

    return out


# speedup vs baseline: 1.4230x; 1.0757x over previous
import jax
import jax.numpy as jnp
import numpy as np
from jax.experimental import pallas as pl
from jax.experimental.pallas import tpu as pltpu

B, L, N, D = 32, 96, 16, 64
E, K = 8, 2
D_FF = 128
LN = L * N
EPS = float(np.finfo(float).eps)


def _body(x_ref, wv_ref, sb_ref, wg_ref, w1_ref, b1_ref, w2_ref, b2_ref,
          o_ref, w2d_ref):
    b = pl.program_id(0)
    xb = x_ref[0]

    t = jnp.sum(xb * wv_ref[...], axis=(1, 2))
    s = t.reshape(L, 1) + sb_ref[...]
    logits = jnp.sum(s * wg_ref[...], axis=0, keepdims=True)
    iota = jax.lax.broadcasted_iota(jnp.int32, (1, E), 1)
    m1 = jnp.max(logits, axis=1, keepdims=True)
    i1 = jnp.min(jnp.where(logits == m1, iota, E), axis=1, keepdims=True)
    l2 = jnp.where(iota == i1, -jnp.inf, logits)
    m2 = jnp.max(l2, axis=1, keepdims=True)
    i2 = jnp.min(jnp.where(l2 == m2, iota, E), axis=1, keepdims=True)
    r = jnp.exp(m2 - m1)
    g1 = 1.0 / (1.0 + r)
    g2 = r / (1.0 + r)
    e1 = i1[0, 0]
    e2 = i2[0, 0]

    w1a = w1_ref[pl.ds(e1, 1)][0]
    w1b = w1_ref[pl.ds(e2, 1)][0]
    b1a = b1_ref[pl.ds(e1, 1)][0]
    b1b = b1_ref[pl.ds(e2, 1)][0]
    b2a = b2_ref[pl.ds(e1, 1)][0]
    b2b = b2_ref[pl.ds(e2, 1)][0]

    xm = xb.reshape(LN, D)
    w1 = jnp.concatenate([w1a, w1b], axis=1)
    bias1 = jnp.concatenate([b1a, b1b], axis=1)
    h = jnp.dot(xm, w1, preferred_element_type=jnp.float32)
    h = jnp.maximum(h + bias1, 0.0)

    @pl.when(b == 0)
    def _():
        w2d_ref[:D_FF, D:] = jnp.zeros((D_FF, D), jnp.float32)
        w2d_ref[D_FF:, :D] = jnp.zeros((D_FF, D), jnp.float32)

    w2d_ref[:D_FF, :D] = w2_ref[pl.ds(e1, 1)][0]
    w2d_ref[D_FF:, D:] = w2_ref[pl.ds(e2, 1)][0]
    bias2 = jnp.concatenate([b2a, b2b], axis=1)
    y = jnp.dot(h, w2d_ref[...], preferred_element_type=jnp.float32)
    y = y + bias2

    ey = jnp.exp(y)
    acc = g1 * ey[:, :D] + g2 * ey[:, D:]
    acc = jnp.where(acc == 0.0, EPS, acc)
    o_ref[0] = jnp.log(acc).reshape(L, N, D)


@jax.jit
def kernel(x, start_w, start_b, w_gate, W1, b1, W2, b2):
    wv = (start_w[:, 0] / N).reshape(1, 1, D)
    sb = start_b.reshape(1, 1)

    out = pl.pallas_call(
        _body,
        grid=(B,),
        in_specs=[
            pl.BlockSpec((1, L, N, D), lambda b: (b, 0, 0, 0)),
            pl.BlockSpec((1, 1, D), lambda b: (0, 0, 0)),
            pl.BlockSpec((1, 1), lambda b: (0, 0)),
            pl.BlockSpec((L, E), lambda b: (0, 0)),
            pl.BlockSpec((E, D, D_FF), lambda b: (0, 0, 0)),
            pl.BlockSpec((E, 1, D_FF), lambda b: (0, 0, 0)),
            pl.BlockSpec((E, D_FF, D), lambda b: (0, 0, 0)),
            pl.BlockSpec((E, 1, D), lambda b: (0, 0, 0)),
        ],
        out_specs=pl.BlockSpec((1, L, N, D), lambda b: (b, 0, 0, 0)),
        out_shape=jax.ShapeDtypeStruct((B, L, N, D), jnp.float32),
        scratch_shapes=[pltpu.VMEM((2 * D_FF, 2 * D), jnp.float32)],
    )(x, wv, sb, w_gate, W1, b1.reshape(E, 1, D_FF), W2,
      b2.reshape(E, 1, D))

    return out


# speedup vs baseline: 1.6954x; 1.1914x over previous
import jax
import jax.numpy as jnp
import numpy as np
from jax.experimental import pallas as pl
from jax.experimental.pallas import tpu as pltpu

B, L, N, D = 32, 96, 16, 64
E, K = 8, 2
D_FF = 128
LN = L * N
EPS = float(np.finfo(float).eps)


def _body(x_ref, sw_ref, sb_ref, wgx_ref, w1_ref, b1_ref, w2_ref, b2_ref,
          o_ref):
    xm = x_ref[0].reshape(LN, D)

    p = jnp.dot(xm, sw_ref[...], preferred_element_type=jnp.float32)
    logits = jnp.dot(wgx_ref[...], p, preferred_element_type=jnp.float32)
    logits = logits + sb_ref[...]
    iota = jax.lax.broadcasted_iota(jnp.int32, (E, 1), 0)
    m1 = jnp.max(logits, axis=0, keepdims=True)
    i1 = jnp.min(jnp.where(logits == m1, iota, E), axis=0, keepdims=True)
    l2 = jnp.where(iota == i1, -jnp.inf, logits)
    m2 = jnp.max(l2, axis=0, keepdims=True)
    i2 = jnp.min(jnp.where(l2 == m2, iota, E), axis=0, keepdims=True)
    r = jnp.exp(m2 - m1)
    g1 = 1.0 / (1.0 + r)
    g2 = r / (1.0 + r)
    e1 = i1[0, 0]
    e2 = i2[0, 0]

    w1a = w1_ref[pl.ds(e1, 1)][0]
    w1b = w1_ref[pl.ds(e2, 1)][0]
    b1a = b1_ref[pl.ds(e1, 1)][0]
    b1b = b1_ref[pl.ds(e2, 1)][0]
    w2a = w2_ref[pl.ds(e1, 1)][0]
    w2b = w2_ref[pl.ds(e2, 1)][0]
    b2a = b2_ref[pl.ds(e1, 1)][0]
    b2b = b2_ref[pl.ds(e2, 1)][0]

    w1 = jnp.concatenate([w1a, w1b], axis=1)
    bias1 = jnp.concatenate([b1a, b1b], axis=1)
    h = jnp.dot(xm, w1, preferred_element_type=jnp.float32)
    h = jnp.maximum(h + bias1, 0.0)
    y1 = jnp.dot(h[:, :D_FF], w2a, preferred_element_type=jnp.float32) + b2a
    y2 = jnp.dot(h[:, D_FF:], w2b, preferred_element_type=jnp.float32) + b2b

    acc = g1 * jnp.exp(y1) + g2 * jnp.exp(y2)
    acc = jnp.where(acc == 0.0, EPS, acc)
    o_ref[0] = jnp.log(acc).reshape(L, N, D)


@jax.jit
def kernel(x, start_w, start_b, w_gate, W1, b1, W2, b2):
    wgx = jnp.repeat(w_gate.T / N, N, axis=1)
    sb = start_b.reshape(1, 1)

    out = pl.pallas_call(
        _body,
        grid=(B,),
        in_specs=[
            pl.BlockSpec((1, L, N, D), lambda b: (b, 0, 0, 0)),
            pl.BlockSpec((D, 1), lambda b: (0, 0)),
            pl.BlockSpec((1, 1), lambda b: (0, 0)),
            pl.BlockSpec((E, LN), lambda b: (0, 0)),
            pl.BlockSpec((E, D, D_FF), lambda b: (0, 0, 0)),
            pl.BlockSpec((E, 1, D_FF), lambda b: (0, 0, 0)),
            pl.BlockSpec((E, D_FF, D), lambda b: (0, 0, 0)),
            pl.BlockSpec((E, 1, D), lambda b: (0, 0, 0)),
        ],
        out_specs=pl.BlockSpec((1, L, N, D), lambda b: (b, 0, 0, 0)),
        out_shape=jax.ShapeDtypeStruct((B, L, N, D), jnp.float32),
    )(x, start_w, sb, wgx, W1, b1.reshape(E, 1, D_FF), W2,
      b2.reshape(E, 1, D))

    return out
